# baseline (speedup 1.0000x reference)
import functools

import jax
import jax.numpy as jnp
from jax import lax
from jax.experimental import pallas as pl
from jax.experimental.pallas import tpu as pltpu
from jax.experimental.pallas import tpu_sc as plsc

NN = 10000
EE = 320000
L = 16
NS = 16
NC = 2
NW = NC * NS
CH = 96

ETOT = EE + NN
CPT_ES = 108
CPT_FS = 2 * CPT_ES
NCH = CPT_ES * NW
EPAD = NCH * CH
DPT = NCH // NS
NNP = 10112
RPTP = NNP // NS
NR = 80

_MESH = plsc.VectorSubcoreMesh(core_axis_name="c", subcore_axis_name="s")
_SC_PARAMS = pltpu.CompilerParams(needs_layout_passes=False)


def _newton_rsqrt(d):
    i = plsc.bitcast(d, jnp.int32)
    i = jnp.int32(0x5F3759DF) - lax.shift_right_logical(i, jnp.int32(1))
    y = plsc.bitcast(i, jnp.float32)
    for _ in range(3):
        y = y * (1.5 - 0.5 * d * y * y)
    return y


def _rc(i16):
    return (lax.shift_right_logical(i16, jnp.int32(7)),
            jnp.bitwise_and(i16, jnp.int32(127)))


def _make_spmm(feat_split, fused_deg):
    cpt = CPT_FS if feat_split else CPT_ES
    ni = cpt // 2

    agg_ty = jax.ShapeDtypeStruct((2 * NNP, 128), jnp.float32)
    if fused_deg:
        out_ty = (agg_ty, jax.ShapeDtypeStruct((NR, 128), jnp.float32))
    else:
        out_ty = agg_ty

    scratch = [
        pltpu.VMEM_SHARED((NNP, 128), jnp.float32),
        pltpu.VMEM((CH, 128), jnp.float32),
        pltpu.VMEM((CH, 128), jnp.float32),
        pltpu.VMEM((CH, 128), jnp.float32),
        pltpu.VMEM((NR, 128), jnp.float32),
        pltpu.VMEM((CH,), jnp.int32),
        pltpu.VMEM((CH,), jnp.int32),
        pltpu.VMEM((CH,), jnp.int32),
        pltpu.VMEM((CH,), jnp.int32),
        pltpu.VMEM((CH,), jnp.int32),
        pltpu.VMEM((CH,), jnp.int32),
        pltpu.VMEM((CH,), jnp.float32),
        pltpu.VMEM((CH,), jnp.float32),
        pltpu.VMEM((CH,), jnp.float32),
        pltpu.VMEM((CH,), jnp.float32),
        pltpu.VMEM((CH,), jnp.float32),
        pltpu.VMEM((CH,), jnp.float32),
        pltpu.SemaphoreType.DMA,
        pltpu.SemaphoreType.DMA,
        pltpu.SemaphoreType.DMA,
        pltpu.SemaphoreType.DMA,
        pltpu.SemaphoreType.DMA,
        pltpu.SemaphoreType.DMA,
        pltpu.SemaphoreType.DMA,
        pltpu.SemaphoreType.DMA,
        pltpu.SemaphoreType.DMA,
    ]
    if fused_deg:
        scratch += [
            pltpu.VMEM_SHARED((NR, 128), jnp.float32),
            pltpu.VMEM((NR,), jnp.int32),
        ]

    @functools.partial(
        pl.kernel,
        out_type=out_ty,
        mesh=_MESH,
        scratch_types=scratch,
        compiler_params=_SC_PARAMS,
    )
    def _spmm(table, srcp, dstp, wp, *rest):
        if fused_deg:
            (agg_out, dinv_out, acc, rows0, rows1, rows2, dlb,
             isv0, isv1, isv2, idv0, idv1, idv2, wv0, wv1, wv2,
             nv0, nv1, nv2, semi0, semi1, semi2, semb0, semb1, semb2,
             sems0, sems1, sems2, sdeg, rix) = rest
        else:
            (dinvh, agg_out, acc, rows0, rows1, rows2, dlb,
             isv0, isv1, isv2, idv0, idv1, idv2, wv0, wv1, wv2,
             nv0, nv1, nv2, semi0, semi1, semi2, semb0, semb1, semb2,
             sems0, sems1, sems2) = rest
        c = lax.axis_index("c")
        s = lax.axis_index("s")
        zrow = jnp.zeros((L,), jnp.float32)

        def zb(j, carry):
            for v in range(128 // L):
                rows0[j, pl.ds(v * L, L)] = zrow
            return carry

        lax.fori_loop(0, CH, zb, 0)
        r0 = s * RPTP
        for k in range(RPTP // CH):
            pltpu.sync_copy(rows0, acc.at[pl.ds(r0 + k * CH, CH)])
        rem = RPTP % CH
        if rem:
            pltpu.sync_copy(rows0.at[pl.ds(0, rem)],
                            acc.at[pl.ds(r0 + (RPTP // CH) * CH, rem)])

        if fused_deg:
            @pl.when(s < NR // 8)
            def _():
                pltpu.sync_copy(rows0.at[pl.ds(0, 8)],
                                sdeg.at[pl.ds(8 * s, 8)])

            def zd(j, carry):
                for v in range(128 // L):
                    dlb[j, pl.ds(v * L, L)] = zrow
                return carry

            lax.fori_loop(0, NR, zd, 0)
            for i in range(NR // L):
                rix[pl.ds(i * L, L)] = lax.iota(jnp.int32, L) + (i * L)
            plsc.subcore_barrier()

            def issue_dw(k, p):
                idv, wv, semi = (idv0, wv0, semi0) if p == 0 else (idv1, wv1, semi1)
                base = k * CH
                pltpu.async_copy(dstp.at[pl.ds(base, CH)], idv, semi)
                pltpu.async_copy(wp.at[pl.ds(base, CH)], wv, semi)

            def wait_dw(p):
                idv, wv, semi = (idv0, wv0, semi0) if p == 0 else (idv1, wv1, semi1)
                pltpu.make_async_copy(dstp.at[pl.ds(0, CH)], idv, semi).wait()
                pltpu.make_async_copy(wp.at[pl.ds(0, CH)], wv, semi).wait()

            def hist(p):
                idv, wv = (idv0, wv0) if p == 0 else (idv1, wv1)
                for v in range(CH // L):
                    sl = pl.ds(v * L, L)
                    r16, c16 = _rc(idv[sl])
                    plsc.addupdate_scatter(dlb, [r16, c16], wv[sl])

            d0 = s * DPT
            issue_dw(d0, 0)

            def dstep(i, carry):
                k0 = d0 + 2 * i
                issue_dw(k0 + 1, 1)
                wait_dw(0)
                hist(0)

                @pl.when(i < DPT // 2 - 1)
                def _():
                    issue_dw(k0 + 2, 0)

                wait_dw(1)
                hist(1)
                return carry

            lax.fori_loop(0, DPT // 2, dstep, 0)
            pltpu.sync_copy(dlb, sdeg.at[rix], add=True)
            plsc.subcore_barrier()

            @pl.when(s < NR // 8)
            def _():
                pltpu.sync_copy(sdeg.at[pl.ds(8 * s, 8)], rows0.at[pl.ds(0, 8)])
                for j in range(8):
                    for v in range(128 // L):
                        sl = pl.ds(v * L, L)
                        d = jnp.maximum(rows0[j, sl], 1.0)
                        rows0[j, sl] = _newton_rsqrt(d)
                pltpu.sync_copy(rows0.at[pl.ds(0, 8)], sdeg.at[pl.ds(8 * s, 8)])

                @pl.when(c == 0)
                def _():
                    pltpu.sync_copy(rows0.at[pl.ds(0, 8)],
                                    dinv_out.at[pl.ds(8 * s, 8)])

            plsc.subcore_barrier()
            pltpu.sync_copy(sdeg, dlb)
        else:
            pltpu.sync_copy(dinvh, dlb)
            plsc.subcore_barrier()

        g0 = s * cpt if feat_split else (c * NS + s) * cpt
        off = c * NN
        bufs = ((isv0, idv0, wv0, nv0, rows0, semi0, semb0, sems0),
                (isv1, idv1, wv1, nv1, rows1, semi1, semb1, sems1),
                (isv2, idv2, wv2, nv2, rows2, semi2, semb2, sems2))

        def issue_idx(k, p):
            isv, idv, wv, _, _, semi, _, _ = bufs[p]
            base = k * CH
            pltpu.async_copy(srcp.at[pl.ds(base, CH)], isv, semi)
            pltpu.async_copy(dstp.at[pl.ds(base, CH)], idv, semi)
            pltpu.async_copy(wp.at[pl.ds(base, CH)], wv, semi)

        def wait_idx(p):
            isv, idv, wv, _, _, semi, _, _ = bufs[p]
            pltpu.make_async_copy(srcp.at[pl.ds(0, CH)], isv, semi).wait()
            pltpu.make_async_copy(dstp.at[pl.ds(0, CH)], idv, semi).wait()
            pltpu.make_async_copy(wp.at[pl.ds(0, CH)], wv, semi).wait()

        def start_gather(p):
            isv, idv, wv, nv, rows, _, semb, _ = bufs[p]
            for v in range(CH // L):
                sl = pl.ds(v * L, L)
                s16 = isv[sl]
                rs, cs = _rc(s16)
                rd, cd = _rc(idv[sl])
                a = plsc.load_gather(dlb, [rs, cs])
                b = plsc.load_gather(dlb, [rd, cd])
                nv[sl] = a * wv[sl] * b
                if feat_split:
                    isv[sl] = s16 + off
            pltpu.async_copy(table.at[isv], rows, semb)

        def wait_gather(p):
            isv, _, _, _, rows, _, semb, _ = bufs[p]
            pltpu.make_async_copy(table.at[isv], rows, semb).wait()

        def scale_rows(p):
            _, _, _, nv, rows, _, _, _ = bufs[p]

            def scale(q, carry2):
                n16 = nv[pl.ds(q * L, L)]
                j0 = q * L
                for e in range(L):
                    nj = n16[e]
                    for v in range(128 // L):
                        sl = pl.ds(v * L, L)
                        rows[j0 + e, sl] = rows[j0 + e, sl] * nj
                return carry2

            lax.fori_loop(0, CH // L, scale, 0)

        def scatter_start(p):
            _, idv, _, _, rows, _, _, sems = bufs[p]
            pltpu.async_copy(rows, acc.at[idv], sems, add=True)

        def scatter_wait(p):
            _, idv, _, _, rows, _, _, sems = bufs[p]
            pltpu.make_async_copy(rows, acc.at[idv], sems).wait()

        ni3 = cpt // 3
        issue_idx(g0, 0)
        wait_idx(0)
        start_gather(0)
        issue_idx(g0 + 1, 1)

        def step(i, carry):
            k0 = g0 + 3 * i
            wait_idx(1)
            start_gather(1)
            wait_gather(0)
            scale_rows(0)

            @pl.when(i > 0)
            def _():
                scatter_wait(2)

            issue_idx(k0 + 2, 2)
            scatter_start(0)

            wait_idx(2)
            start_gather(2)
            wait_gather(1)
            scale_rows(1)
            scatter_wait(0)

            @pl.when(i < ni3 - 1)
            def _():
                issue_idx(k0 + 3, 0)

            scatter_start(1)

            @pl.when(i < ni3 - 1)
            def _():
                wait_idx(0)
                start_gather(0)

            wait_gather(2)
            scale_rows(2)
            scatter_wait(1)

            @pl.when(i < ni3 - 1)
            def _():
                issue_idx(k0 + 4, 1)

            scatter_start(2)
            return carry

        lax.fori_loop(0, ni3, step, 0)
        scatter_wait(2)
        plsc.subcore_barrier()
        pltpu.sync_copy(acc.at[pl.ds(r0, RPTP)],
                        agg_out.at[pl.ds(c * NNP + r0, RPTP)])

    return _spmm


_scf1 = _make_spmm(False, True)
_spmm_fs = _make_spmm(True, False)
_spmm_es = _make_spmm(False, False)


RB = 400
GRID = NN // RB


def _l1_body(agg_ref, w1_ref, b1_ref, w2_ref, out_ref):
    a = agg_ref[0] + agg_ref[1]
    h = jnp.dot(a, w1_ref[...], preferred_element_type=jnp.float32) + b1_ref[...]
    h = jnp.maximum(h, 0.0)
    t = jnp.dot(h, w2_ref[...], preferred_element_type=jnp.float32)
    out_ref[0] = t[:, :128]
    out_ref[1] = t[:, 128:]


_l1_call = pl.pallas_call(
    _l1_body,
    grid=(GRID,),
    in_specs=[
        pl.BlockSpec((2, RB, 128), lambda i: (0, i, 0)),
        pl.BlockSpec((128, 512), lambda i: (0, 0)),
        pl.BlockSpec((1, 512), lambda i: (0, 0)),
        pl.BlockSpec((512, 256), lambda i: (0, 0)),
    ],
    out_specs=pl.BlockSpec((2, RB, 128), lambda i: (0, i, 0)),
    out_shape=jax.ShapeDtypeStruct((2, NN, 128), jnp.float32),
)


def _l2_body(agg_ref, b2_ref, w3_ref, out_ref):
    h = jnp.concatenate([agg_ref[0], agg_ref[1]], axis=1) + b2_ref[...]
    h = jnp.maximum(h, 0.0)
    out_ref[...] = jnp.dot(h, w3_ref[...], preferred_element_type=jnp.float32)


_l2_call = pl.pallas_call(
    _l2_body,
    grid=(GRID,),
    in_specs=[
        pl.BlockSpec((2, RB, 128), lambda i: (0, i, 0)),
        pl.BlockSpec((1, 256), lambda i: (0, 0)),
        pl.BlockSpec((256, 128), lambda i: (0, 0)),
    ],
    out_specs=pl.BlockSpec((RB, 128), lambda i: (i, 0)),
    out_shape=jax.ShapeDtypeStruct((NN, 128), jnp.float32),
)


def _l3_body(agg_ref, b3_ref, out_ref):
    out_ref[...] = agg_ref[0] + agg_ref[1] + b3_ref[...]


_l3_call = pl.pallas_call(
    _l3_body,
    grid=(GRID,),
    in_specs=[
        pl.BlockSpec((2, RB, 128), lambda i: (0, i, 0)),
        pl.BlockSpec((1, 128), lambda i: (0, 0)),
    ],
    out_specs=pl.BlockSpec((RB, 128), lambda i: (i, 0)),
    out_shape=jax.ShapeDtypeStruct((NN, 128), jnp.float32),
)


def kernel(x, edge_index, edge_attr, W1, b1, W2, b2, W3, b3):
    src = edge_index[0]
    dst = edge_index[1]
    loop = jnp.arange(NN, dtype=jnp.int32)
    npad = EPAD - ETOT
    padi = jnp.arange(npad, dtype=jnp.int32) % NN
    srcp = jnp.concatenate([src, loop, padi])
    dstp = jnp.concatenate([dst, loop, padi])
    wp = jnp.concatenate(
        [edge_attr, jnp.ones((NN,), jnp.float32), jnp.zeros((npad,), jnp.float32)])

    agg1, dinvh = _scf1(x, srcp, dstp, wp)
    t2 = _l1_call(agg1.reshape(2, NNP, 128), W1, b1.reshape(1, 512), W2)
    agg2 = _spmm_fs(t2.reshape(2 * NN, 128), srcp, dstp, wp, dinvh)
    t3 = _l2_call(agg2.reshape(2, NNP, 128), b2.reshape(1, 256), W3)
    agg3 = _spmm_es(t3, srcp, dstp, wp, dinvh)
    out = _l3_call(agg3.reshape(2, NNP, 128), b3.reshape(1, 128))
    return out


# speedup vs baseline: 1.0421x; 1.0421x over previous
import functools

import jax
import jax.numpy as jnp
from jax import lax
from jax.experimental import pallas as pl
from jax.experimental.pallas import tpu as pltpu
from jax.experimental.pallas import tpu_sc as plsc

NN = 10000
EE = 320000
L = 16
NS = 16
NC = 2
NW = NC * NS
CH = 128

ETOT = EE + NN
CPT_ES = 82
CPT_FS = 2 * CPT_ES
NCH = CPT_ES * NW
EPAD = NCH * CH
DPT = NCH // NS
NNP = 10112
RPTP = NNP // NS
NR = 80

_MESH = plsc.VectorSubcoreMesh(core_axis_name="c", subcore_axis_name="s")
_SC_PARAMS = pltpu.CompilerParams(needs_layout_passes=False)


def _newton_rsqrt(d):
    i = plsc.bitcast(d, jnp.int32)
    i = jnp.int32(0x5F3759DF) - lax.shift_right_logical(i, jnp.int32(1))
    y = plsc.bitcast(i, jnp.float32)
    for _ in range(3):
        y = y * (1.5 - 0.5 * d * y * y)
    return y


def _rc(i16):
    return (lax.shift_right_logical(i16, jnp.int32(7)),
            jnp.bitwise_and(i16, jnp.int32(127)))


def _make_spmm(feat_split, fused_deg):
    cpt = CPT_FS if feat_split else CPT_ES
    ni = cpt // 2

    agg_ty = jax.ShapeDtypeStruct((2 * NNP, 128), jnp.float32)
    if fused_deg:
        out_ty = (agg_ty, jax.ShapeDtypeStruct((NR, 128), jnp.float32))
    else:
        out_ty = agg_ty

    scratch = [
        pltpu.VMEM_SHARED((NNP, 128), jnp.float32),
        pltpu.VMEM((CH, 128), jnp.float32),
        pltpu.VMEM((CH, 128), jnp.float32),
        pltpu.VMEM((NR, 128), jnp.float32),
        pltpu.VMEM((CH,), jnp.int32),
        pltpu.VMEM((CH,), jnp.int32),
        pltpu.VMEM((CH,), jnp.int32),
        pltpu.VMEM((CH,), jnp.int32),
        pltpu.VMEM((CH,), jnp.float32),
        pltpu.VMEM((CH,), jnp.float32),
        pltpu.VMEM((CH,), jnp.float32),
        pltpu.VMEM((CH,), jnp.float32),
        pltpu.SemaphoreType.DMA,
        pltpu.SemaphoreType.DMA,
        pltpu.SemaphoreType.DMA,
        pltpu.SemaphoreType.DMA,
    ]
    if fused_deg:
        scratch += [
            pltpu.VMEM_SHARED((NR, 128), jnp.float32),
            pltpu.VMEM((NR,), jnp.int32),
        ]

    @functools.partial(
        pl.kernel,
        out_type=out_ty,
        mesh=_MESH,
        scratch_types=scratch,
        compiler_params=_SC_PARAMS,
    )
    def _spmm(table, srcp, dstp, wp, *rest):
        if fused_deg:
            (agg_out, dinv_out, acc, rows0, rows1, dlb,
             isv0, isv1, idv0, idv1, wv0, wv1, nv0, nv1,
             semi0, semi1, semb0, semb1, sdeg, rix) = rest
        else:
            (dinvh, agg_out, acc, rows0, rows1, dlb,
             isv0, isv1, idv0, idv1, wv0, wv1, nv0, nv1,
             semi0, semi1, semb0, semb1) = rest
        c = lax.axis_index("c")
        s = lax.axis_index("s")
        zrow = jnp.zeros((L,), jnp.float32)

        def zb(j, carry):
            for v in range(128 // L):
                rows0[j, pl.ds(v * L, L)] = zrow
            return carry

        lax.fori_loop(0, CH, zb, 0)
        r0 = s * RPTP
        for k in range(RPTP // CH):
            pltpu.sync_copy(rows0, acc.at[pl.ds(r0 + k * CH, CH)])
        rem = RPTP % CH
        if rem:
            pltpu.sync_copy(rows0.at[pl.ds(0, rem)],
                            acc.at[pl.ds(r0 + (RPTP // CH) * CH, rem)])

        if fused_deg:
            @pl.when(s < NR // 8)
            def _():
                pltpu.sync_copy(rows0.at[pl.ds(0, 8)],
                                sdeg.at[pl.ds(8 * s, 8)])

            def zd(j, carry):
                for v in range(128 // L):
                    dlb[j, pl.ds(v * L, L)] = zrow
                return carry

            lax.fori_loop(0, NR, zd, 0)
            for i in range(NR // L):
                rix[pl.ds(i * L, L)] = lax.iota(jnp.int32, L) + (i * L)
            plsc.subcore_barrier()

            def issue_dw(k, p):
                idv, wv, semi = (idv0, wv0, semi0) if p == 0 else (idv1, wv1, semi1)
                base = k * CH
                pltpu.async_copy(dstp.at[pl.ds(base, CH)], idv, semi)
                pltpu.async_copy(wp.at[pl.ds(base, CH)], wv, semi)

            def wait_dw(p):
                idv, wv, semi = (idv0, wv0, semi0) if p == 0 else (idv1, wv1, semi1)
                pltpu.make_async_copy(dstp.at[pl.ds(0, CH)], idv, semi).wait()
                pltpu.make_async_copy(wp.at[pl.ds(0, CH)], wv, semi).wait()

            def hist(p):
                idv, wv = (idv0, wv0) if p == 0 else (idv1, wv1)
                for v in range(CH // L):
                    sl = pl.ds(v * L, L)
                    r16, c16 = _rc(idv[sl])
                    plsc.addupdate_scatter(dlb, [r16, c16], wv[sl])

            d0 = s * DPT
            issue_dw(d0, 0)

            def dstep(i, carry):
                k0 = d0 + 2 * i
                issue_dw(k0 + 1, 1)
                wait_dw(0)
                hist(0)

                @pl.when(i < DPT // 2 - 1)
                def _():
                    issue_dw(k0 + 2, 0)

                wait_dw(1)
                hist(1)
                return carry

            lax.fori_loop(0, DPT // 2, dstep, 0)
            pltpu.sync_copy(dlb, sdeg.at[rix], add=True)
            plsc.subcore_barrier()

            @pl.when(s < NR // 8)
            def _():
                pltpu.sync_copy(sdeg.at[pl.ds(8 * s, 8)], rows0.at[pl.ds(0, 8)])
                for j in range(8):
                    for v in range(128 // L):
                        sl = pl.ds(v * L, L)
                        d = jnp.maximum(rows0[j, sl], 1.0)
                        rows0[j, sl] = _newton_rsqrt(d)
                pltpu.sync_copy(rows0.at[pl.ds(0, 8)], sdeg.at[pl.ds(8 * s, 8)])

                @pl.when(c == 0)
                def _():
                    pltpu.sync_copy(rows0.at[pl.ds(0, 8)],
                                    dinv_out.at[pl.ds(8 * s, 8)])

            plsc.subcore_barrier()
            pltpu.sync_copy(sdeg, dlb)
        else:
            pltpu.sync_copy(dinvh, dlb)
            plsc.subcore_barrier()

        g0 = s * cpt if feat_split else (c * NS + s) * cpt
        off = c * NN
        bufs = ((isv0, idv0, wv0, nv0, rows0, semi0, semb0),
                (isv1, idv1, wv1, nv1, rows1, semi1, semb1))

        def issue_idx(k, p):
            isv, idv, wv, _, _, semi, _ = bufs[p]
            base = k * CH
            pltpu.async_copy(srcp.at[pl.ds(base, CH)], isv, semi)
            pltpu.async_copy(dstp.at[pl.ds(base, CH)], idv, semi)
            pltpu.async_copy(wp.at[pl.ds(base, CH)], wv, semi)

        def wait_idx(p):
            isv, idv, wv, _, _, semi, _ = bufs[p]
            pltpu.make_async_copy(srcp.at[pl.ds(0, CH)], isv, semi).wait()
            pltpu.make_async_copy(dstp.at[pl.ds(0, CH)], idv, semi).wait()
            pltpu.make_async_copy(wp.at[pl.ds(0, CH)], wv, semi).wait()

        def start_gather(p):
            isv, idv, wv, nv, rows, _, semb = bufs[p]
            for v in range(CH // L):
                sl = pl.ds(v * L, L)
                s16 = isv[sl]
                rs, cs = _rc(s16)
                rd, cd = _rc(idv[sl])
                a = plsc.load_gather(dlb, [rs, cs])
                b = plsc.load_gather(dlb, [rd, cd])
                nv[sl] = a * wv[sl] * b
                if feat_split:
                    isv[sl] = s16 + off
            pltpu.async_copy(table.at[isv], rows, semb)

        def wait_gather(p):
            isv, _, _, _, rows, _, semb = bufs[p]
            pltpu.make_async_copy(table.at[isv], rows, semb).wait()

        def scale_rows(p):
            _, _, _, nv, rows, _, _ = bufs[p]

            def scale(q, carry2):
                n16 = nv[pl.ds(q * L, L)]
                j0 = q * L
                for e in range(L):
                    nj = n16[e]
                    for v in range(128 // L):
                        sl = pl.ds(v * L, L)
                        rows[j0 + e, sl] = rows[j0 + e, sl] * nj
                return carry2

            lax.fori_loop(0, CH // L, scale, 0)

        def scatter(p):
            _, idv, _, _, rows, _, _ = bufs[p]
            pltpu.sync_copy(rows, acc.at[idv], add=True)

        issue_idx(g0, 0)
        wait_idx(0)
        start_gather(0)
        issue_idx(g0 + 1, 1)

        def step(i, carry):
            k0 = g0 + 2 * i
            wait_idx(1)
            start_gather(1)
            wait_gather(0)
            scale_rows(0)

            @pl.when(i < ni - 1)
            def _():
                issue_idx(k0 + 2, 0)

            scatter(0)

            @pl.when(i < ni - 1)
            def _():
                wait_idx(0)
                start_gather(0)

            wait_gather(1)
            scale_rows(1)

            @pl.when(i < ni - 1)
            def _():
                issue_idx(k0 + 3, 1)

            scatter(1)
            return carry

        lax.fori_loop(0, ni, step, 0)
        plsc.subcore_barrier()
        pltpu.sync_copy(acc.at[pl.ds(r0, RPTP)],
                        agg_out.at[pl.ds(c * NNP + r0, RPTP)])

    return _spmm


_scf1 = _make_spmm(False, True)
_spmm_fs = _make_spmm(True, False)
_spmm_es = _make_spmm(False, False)


RB = 400
GRID = NN // RB


def _l1_body(agg_ref, w1_ref, b1_ref, w2_ref, out_ref):
    a = agg_ref[0] + agg_ref[1]
    h = jnp.dot(a, w1_ref[...], preferred_element_type=jnp.float32) + b1_ref[...]
    h = jnp.maximum(h, 0.0)
    t = jnp.dot(h, w2_ref[...], preferred_element_type=jnp.float32)
    out_ref[0] = t[:, :128]
    out_ref[1] = t[:, 128:]


_l1_call = pl.pallas_call(
    _l1_body,
    grid=(GRID,),
    in_specs=[
        pl.BlockSpec((2, RB, 128), lambda i: (0, i, 0)),
        pl.BlockSpec((128, 512), lambda i: (0, 0)),
        pl.BlockSpec((1, 512), lambda i: (0, 0)),
        pl.BlockSpec((512, 256), lambda i: (0, 0)),
    ],
    out_specs=pl.BlockSpec((2, RB, 128), lambda i: (0, i, 0)),
    out_shape=jax.ShapeDtypeStruct((2, NN, 128), jnp.float32),
)


def _l2_body(agg_ref, b2_ref, w3_ref, out_ref):
    h = jnp.concatenate([agg_ref[0], agg_ref[1]], axis=1) + b2_ref[...]
    h = jnp.maximum(h, 0.0)
    out_ref[...] = jnp.dot(h, w3_ref[...], preferred_element_type=jnp.float32)


_l2_call = pl.pallas_call(
    _l2_body,
    grid=(GRID,),
    in_specs=[
        pl.BlockSpec((2, RB, 128), lambda i: (0, i, 0)),
        pl.BlockSpec((1, 256), lambda i: (0, 0)),
        pl.BlockSpec((256, 128), lambda i: (0, 0)),
    ],
    out_specs=pl.BlockSpec((RB, 128), lambda i: (i, 0)),
    out_shape=jax.ShapeDtypeStruct((NN, 128), jnp.float32),
)


def _l3_body(agg_ref, b3_ref, out_ref):
    out_ref[...] = agg_ref[0] + agg_ref[1] + b3_ref[...]


_l3_call = pl.pallas_call(
    _l3_body,
    grid=(GRID,),
    in_specs=[
        pl.BlockSpec((2, RB, 128), lambda i: (0, i, 0)),
        pl.BlockSpec((1, 128), lambda i: (0, 0)),
    ],
    out_specs=pl.BlockSpec((RB, 128), lambda i: (i, 0)),
    out_shape=jax.ShapeDtypeStruct((NN, 128), jnp.float32),
)


def kernel(x, edge_index, edge_attr, W1, b1, W2, b2, W3, b3):
    src = edge_index[0]
    dst = edge_index[1]
    loop = jnp.arange(NN, dtype=jnp.int32)
    npad = EPAD - ETOT
    padi = jnp.arange(npad, dtype=jnp.int32) % NN
    srcp = jnp.concatenate([src, loop, padi])
    dstp = jnp.concatenate([dst, loop, padi])
    wp = jnp.concatenate(
        [edge_attr, jnp.ones((NN,), jnp.float32), jnp.zeros((npad,), jnp.float32)])

    agg1, dinvh = _scf1(x, srcp, dstp, wp)
    t2 = _l1_call(agg1.reshape(2, NNP, 128), W1, b1.reshape(1, 512), W2)
    agg2 = _spmm_fs(t2.reshape(2 * NN, 128), srcp, dstp, wp, dinvh)
    t3 = _l2_call(agg2.reshape(2, NNP, 128), b2.reshape(1, 256), W3)
    agg3 = _spmm_es(t3, srcp, dstp, wp, dinvh)
    out = _l3_call(agg3.reshape(2, NNP, 128), b3.reshape(1, 128))
    return out


# speedup vs baseline: 1.0454x; 1.0032x over previous
import functools

import jax
import jax.numpy as jnp
from jax import lax
from jax.experimental import pallas as pl
from jax.experimental.pallas import tpu as pltpu
from jax.experimental.pallas import tpu_sc as plsc

NN = 10000
EE = 320000
L = 16
NS = 16
NC = 2
NW = NC * NS
CH = 128

ETOT = EE + NN
CPT_ES = 82
CPT_FS = 2 * CPT_ES
NCH = CPT_ES * NW
EPAD = NCH * CH
DPT = NCH // NS
NNP = 10112
RPTP = NNP // NS
NR = 80

_MESH = plsc.VectorSubcoreMesh(core_axis_name="c", subcore_axis_name="s")
_SC_PARAMS = pltpu.CompilerParams(needs_layout_passes=False)


def _newton_rsqrt(d):
    i = plsc.bitcast(d, jnp.int32)
    i = jnp.int32(0x5F3759DF) - lax.shift_right_logical(i, jnp.int32(1))
    y = plsc.bitcast(i, jnp.float32)
    for _ in range(3):
        y = y * (1.5 - 0.5 * d * y * y)
    return y


def _rc(i16):
    return (lax.shift_right_logical(i16, jnp.int32(7)),
            jnp.bitwise_and(i16, jnp.int32(127)))


def _make_spmm(feat_split, fused_deg):
    cpt = CPT_FS if feat_split else CPT_ES
    ni = cpt // 2

    agg_ty = jax.ShapeDtypeStruct((2 * NNP, 128), jnp.float32)
    if fused_deg:
        out_ty = (agg_ty, jax.ShapeDtypeStruct((NR, 128), jnp.float32))
    else:
        out_ty = agg_ty

    scratch = [
        pltpu.VMEM_SHARED((NNP, 128), jnp.float32),
        pltpu.VMEM((CH, 128), jnp.float32),
        pltpu.VMEM((CH, 128), jnp.float32),
        pltpu.VMEM((NR, 128), jnp.float32),
        pltpu.VMEM((CH,), jnp.int32),
        pltpu.VMEM((CH,), jnp.int32),
        pltpu.VMEM((CH,), jnp.int32),
        pltpu.VMEM((CH,), jnp.int32),
        pltpu.VMEM((CH,), jnp.float32),
        pltpu.VMEM((CH,), jnp.float32),
        pltpu.VMEM((CH,), jnp.float32),
        pltpu.VMEM((CH,), jnp.float32),
        pltpu.SemaphoreType.DMA,
        pltpu.SemaphoreType.DMA,
        pltpu.SemaphoreType.DMA,
        pltpu.SemaphoreType.DMA,
    ]
    if fused_deg:
        scratch += [
            pltpu.VMEM_SHARED((NR, 128), jnp.float32),
            pltpu.VMEM((NR,), jnp.int32),
        ]

    @functools.partial(
        pl.kernel,
        out_type=out_ty,
        mesh=_MESH,
        scratch_types=scratch,
        compiler_params=_SC_PARAMS,
    )
    def _spmm(table, srcp, dstp, wp, *rest):
        if fused_deg:
            (agg_out, dinv_out, acc, rows0, rows1, dlb,
             isv0, isv1, idv0, idv1, wv0, wv1, nv0, nv1,
             semi0, semi1, semb0, semb1, sdeg, rix) = rest
        else:
            (dinvh, agg_out, acc, rows0, rows1, dlb,
             isv0, isv1, idv0, idv1, wv0, wv1, nv0, nv1,
             semi0, semi1, semb0, semb1) = rest
        c = lax.axis_index("c")
        s = lax.axis_index("s")
        zrow = jnp.zeros((L,), jnp.float32)

        def zb(j, carry):
            for v in range(128 // L):
                rows0[j, pl.ds(v * L, L)] = zrow
            return carry

        lax.fori_loop(0, CH, zb, 0)
        r0 = s * RPTP
        for k in range(RPTP // CH):
            pltpu.sync_copy(rows0, acc.at[pl.ds(r0 + k * CH, CH)])
        rem = RPTP % CH
        if rem:
            pltpu.sync_copy(rows0.at[pl.ds(0, rem)],
                            acc.at[pl.ds(r0 + (RPTP // CH) * CH, rem)])

        if fused_deg:
            @pl.when(s < NR // 8)
            def _():
                pltpu.sync_copy(rows0.at[pl.ds(0, 8)],
                                sdeg.at[pl.ds(8 * s, 8)])

            def zd(j, carry):
                for v in range(128 // L):
                    dlb[j, pl.ds(v * L, L)] = zrow
                return carry

            lax.fori_loop(0, NR, zd, 0)
            for i in range(NR // L):
                rix[pl.ds(i * L, L)] = lax.iota(jnp.int32, L) + (i * L)
            plsc.subcore_barrier()

            def issue_dw(k, p):
                idv, wv, semi = (idv0, wv0, semi0) if p == 0 else (idv1, wv1, semi1)
                base = k * CH
                pltpu.async_copy(dstp.at[pl.ds(base, CH)], idv, semi)
                pltpu.async_copy(wp.at[pl.ds(base, CH)], wv, semi)

            def wait_dw(p):
                idv, wv, semi = (idv0, wv0, semi0) if p == 0 else (idv1, wv1, semi1)
                pltpu.make_async_copy(dstp.at[pl.ds(0, CH)], idv, semi).wait()
                pltpu.make_async_copy(wp.at[pl.ds(0, CH)], wv, semi).wait()

            def hist(p):
                idv, wv = (idv0, wv0) if p == 0 else (idv1, wv1)
                for v in range(CH // L):
                    sl = pl.ds(v * L, L)
                    r16, c16 = _rc(idv[sl])
                    plsc.addupdate_scatter(dlb, [r16, c16], wv[sl])

            d0 = s * DPT
            issue_dw(d0, 0)

            def dstep(i, carry):
                k0 = d0 + 2 * i
                issue_dw(k0 + 1, 1)
                wait_dw(0)
                hist(0)

                @pl.when(i < DPT // 2 - 1)
                def _():
                    issue_dw(k0 + 2, 0)

                wait_dw(1)
                hist(1)
                return carry

            lax.fori_loop(0, DPT // 2, dstep, 0)
            pltpu.sync_copy(dlb, sdeg.at[rix], add=True)
            plsc.subcore_barrier()

            @pl.when(s < NR // 8)
            def _():
                pltpu.sync_copy(sdeg.at[pl.ds(8 * s, 8)], rows0.at[pl.ds(0, 8)])
                for j in range(8):
                    for v in range(128 // L):
                        sl = pl.ds(v * L, L)
                        d = jnp.maximum(rows0[j, sl], 1.0)
                        rows0[j, sl] = _newton_rsqrt(d)
                pltpu.sync_copy(rows0.at[pl.ds(0, 8)], sdeg.at[pl.ds(8 * s, 8)])

                @pl.when(c == 0)
                def _():
                    pltpu.sync_copy(rows0.at[pl.ds(0, 8)],
                                    dinv_out.at[pl.ds(8 * s, 8)])

            plsc.subcore_barrier()
            pltpu.sync_copy(sdeg, dlb)
        else:
            pltpu.sync_copy(dinvh, dlb)
            plsc.subcore_barrier()

        g0 = s * cpt if feat_split else (c * NS + s) * cpt
        off = c * NN
        bufs = ((isv0, idv0, wv0, nv0, rows0, semi0, semb0),
                (isv1, idv1, wv1, nv1, rows1, semi1, semb1))

        def issue_idx(k, p):
            isv, idv, wv, _, _, semi, _ = bufs[p]
            base = k * CH
            pltpu.async_copy(srcp.at[pl.ds(base, CH)], isv, semi)
            pltpu.async_copy(dstp.at[pl.ds(base, CH)], idv, semi)
            pltpu.async_copy(wp.at[pl.ds(base, CH)], wv, semi)

        def wait_idx(p):
            isv, idv, wv, _, _, semi, _ = bufs[p]
            pltpu.make_async_copy(srcp.at[pl.ds(0, CH)], isv, semi).wait()
            pltpu.make_async_copy(dstp.at[pl.ds(0, CH)], idv, semi).wait()
            pltpu.make_async_copy(wp.at[pl.ds(0, CH)], wv, semi).wait()

        def start_gather(p):
            isv, idv, wv, nv, rows, _, semb = bufs[p]
            avs = []
            for v in range(CH // L):
                sl = pl.ds(v * L, L)
                s16 = isv[sl]
                rs, cs = _rc(s16)
                avs.append(plsc.load_gather(dlb, [rs, cs]))
                if feat_split:
                    isv[sl] = s16 + off
            pltpu.async_copy(table.at[isv], rows, semb)
            for v in range(CH // L):
                sl = pl.ds(v * L, L)
                rd, cd = _rc(idv[sl])
                b = plsc.load_gather(dlb, [rd, cd])
                nv[sl] = avs[v] * wv[sl] * b

        def wait_gather(p):
            isv, _, _, _, rows, _, semb = bufs[p]
            pltpu.make_async_copy(table.at[isv], rows, semb).wait()

        def scale_rows(p):
            _, _, _, nv, rows, _, _ = bufs[p]

            def scale(q, carry2):
                n16 = nv[pl.ds(q * L, L)]
                j0 = q * L
                for e in range(L):
                    nj = n16[e]
                    for v in range(128 // L):
                        sl = pl.ds(v * L, L)
                        rows[j0 + e, sl] = rows[j0 + e, sl] * nj
                return carry2

            lax.fori_loop(0, CH // L, scale, 0)

        def scatter(p):
            _, idv, _, _, rows, _, _ = bufs[p]
            pltpu.sync_copy(rows, acc.at[idv], add=True)

        issue_idx(g0, 0)
        wait_idx(0)
        start_gather(0)
        issue_idx(g0 + 1, 1)

        def step(i, carry):
            k0 = g0 + 2 * i
            wait_idx(1)
            start_gather(1)
            wait_gather(0)
            scale_rows(0)

            @pl.when(i < ni - 1)
            def _():
                issue_idx(k0 + 2, 0)

            scatter(0)

            @pl.when(i < ni - 1)
            def _():
                wait_idx(0)
                start_gather(0)

            wait_gather(1)
            scale_rows(1)

            @pl.when(i < ni - 1)
            def _():
                issue_idx(k0 + 3, 1)

            scatter(1)
            return carry

        lax.fori_loop(0, ni, step, 0)
        plsc.subcore_barrier()
        pltpu.sync_copy(acc.at[pl.ds(r0, RPTP)],
                        agg_out.at[pl.ds(c * NNP + r0, RPTP)])

    return _spmm


_scf1 = _make_spmm(False, True)
_spmm_fs = _make_spmm(True, False)
_spmm_es = _make_spmm(False, False)


RB = 400
GRID = NN // RB


def _l1_body(agg_ref, w1_ref, b1_ref, w2_ref, out_ref):
    a = agg_ref[0] + agg_ref[1]
    h = jnp.dot(a, w1_ref[...], preferred_element_type=jnp.float32) + b1_ref[...]
    h = jnp.maximum(h, 0.0)
    t = jnp.dot(h, w2_ref[...], preferred_element_type=jnp.float32)
    out_ref[0] = t[:, :128]
    out_ref[1] = t[:, 128:]


_l1_call = pl.pallas_call(
    _l1_body,
    grid=(GRID,),
    in_specs=[
        pl.BlockSpec((2, RB, 128), lambda i: (0, i, 0)),
        pl.BlockSpec((128, 512), lambda i: (0, 0)),
        pl.BlockSpec((1, 512), lambda i: (0, 0)),
        pl.BlockSpec((512, 256), lambda i: (0, 0)),
    ],
    out_specs=pl.BlockSpec((2, RB, 128), lambda i: (0, i, 0)),
    out_shape=jax.ShapeDtypeStruct((2, NN, 128), jnp.float32),
)


def _l2_body(agg_ref, b2_ref, w3_ref, out_ref):
    h = jnp.concatenate([agg_ref[0], agg_ref[1]], axis=1) + b2_ref[...]
    h = jnp.maximum(h, 0.0)
    out_ref[...] = jnp.dot(h, w3_ref[...], preferred_element_type=jnp.float32)


_l2_call = pl.pallas_call(
    _l2_body,
    grid=(GRID,),
    in_specs=[
        pl.BlockSpec((2, RB, 128), lambda i: (0, i, 0)),
        pl.BlockSpec((1, 256), lambda i: (0, 0)),
        pl.BlockSpec((256, 128), lambda i: (0, 0)),
    ],
    out_specs=pl.BlockSpec((RB, 128), lambda i: (i, 0)),
    out_shape=jax.ShapeDtypeStruct((NN, 128), jnp.float32),
)


def _l3_body(agg_ref, b3_ref, out_ref):
    out_ref[...] = agg_ref[0] + agg_ref[1] + b3_ref[...]


_l3_call = pl.pallas_call(
    _l3_body,
    grid=(GRID,),
    in_specs=[
        pl.BlockSpec((2, RB, 128), lambda i: (0, i, 0)),
        pl.BlockSpec((1, 128), lambda i: (0, 0)),
    ],
    out_specs=pl.BlockSpec((RB, 128), lambda i: (i, 0)),
    out_shape=jax.ShapeDtypeStruct((NN, 128), jnp.float32),
)


def kernel(x, edge_index, edge_attr, W1, b1, W2, b2, W3, b3):
    src = edge_index[0]
    dst = edge_index[1]
    loop = jnp.arange(NN, dtype=jnp.int32)
    npad = EPAD - ETOT
    padi = jnp.arange(npad, dtype=jnp.int32) % NN
    srcp = jnp.concatenate([src, loop, padi])
    dstp = jnp.concatenate([dst, loop, padi])
    wp = jnp.concatenate(
        [edge_attr, jnp.ones((NN,), jnp.float32), jnp.zeros((npad,), jnp.float32)])

    agg1, dinvh = _scf1(x, srcp, dstp, wp)
    t2 = _l1_call(agg1.reshape(2, NNP, 128), W1, b1.reshape(1, 512), W2)
    agg2 = _spmm_fs(t2.reshape(2 * NN, 128), srcp, dstp, wp, dinvh)
    t3 = _l2_call(agg2.reshape(2, NNP, 128), b2.reshape(1, 256), W3)
    agg3 = _spmm_es(t3, srcp, dstp, wp, dinvh)
    out = _l3_call(agg3.reshape(2, NNP, 128), b3.reshape(1, 128))
    return out
